# baseline (speedup 1.0000x reference)
import jax
import jax.numpy as jnp
from jax.experimental import pallas as pl
from jax.experimental.pallas import tpu as pltpu

BANK = 512
DIM = 768
EPSV = 1e-6
MOM = 0.9
BLK = 2048
NBLK = 16384 // BLK


def _norm_rows(x):
    n = jnp.sqrt(jnp.sum(x * x, axis=1, keepdims=True))
    return x / jnp.maximum(n, EPSV)


def _body(emb_ref, out_ref, pn_ref, pnb_ref, sums_ref, counts_ref,
          ebp_ref, ohwp_ref):
    i = pl.program_id(0)

    @pl.when(i == 0)
    def _init():
        pn = _norm_rows(_norm_rows(emb_ref[:BANK]))
        pn_ref[...] = pn
        pnb_ref[...] = pn.astype(jnp.bfloat16)
        sums_ref[...] = jnp.zeros_like(sums_ref)
        counts_ref[...] = jnp.zeros_like(counts_ref)
        ebp_ref[...] = jnp.zeros_like(ebp_ref)
        ohwp_ref[...] = jnp.zeros_like(ohwp_ref)

    sums_ref[...] += jax.lax.dot_general(
        ohwp_ref[...], ebp_ref[...], (((0,), (0,)), ((), ())),
        preferred_element_type=jnp.float32,
    )

    e = emb_ref[...]
    eb = e.astype(jnp.bfloat16)
    n2 = jax.lax.dot_general(
        eb * eb, jnp.ones((DIM, 1), jnp.bfloat16), (((1,), (0,)), ((), ())),
        preferred_element_type=jnp.float32,
    )
    w = jax.lax.rsqrt(jnp.maximum(n2, EPSV * EPSV))
    s = jax.lax.dot_general(
        eb, pnb_ref[...], (((1,), (1,)), ((), ())),
        preferred_element_type=jnp.float32,
    )
    m = jnp.max(s, axis=1, keepdims=True)
    first = jnp.where(i == 0, BANK, 0)
    live = jax.lax.broadcasted_iota(jnp.int32, (BLK, 1), 0) >= first
    m = jnp.where(live, m, jnp.inf)
    hit = s >= m
    onehotw = jnp.where(hit, w, 0.0).astype(jnp.bfloat16)
    ebp_ref[...] = eb
    ohwp_ref[...] = onehotw
    cnt = jnp.sum((onehotw > 0).astype(jnp.float32), axis=0, keepdims=True)
    counts_ref[...] += jnp.where(i < NBLK, cnt, 0.0)

    @pl.when(i == NBLK)
    def _fin():
        ident = (jax.lax.broadcasted_iota(jnp.int32, (BANK, BANK), 0)
                 == jax.lax.broadcasted_iota(jnp.int32, (BANK, BANK), 1)
                 ).astype(jnp.float32)
        counts = jax.lax.dot_general(
            ident, counts_ref[...], (((1,), (1,)), ((), ())),
            preferred_element_type=jnp.float32,
        )
        means = sums_ref[...] / jnp.maximum(counts, 1.0)
        pn = pn_ref[...]
        upd = MOM * pn + (1.0 - MOM) * means
        updn = _norm_rows(upd)
        out_ref[...] = jnp.where(counts > 0.0, updn, pn)


def kernel(embeddings):
    emb = embeddings.astype(jnp.float32)
    return pl.pallas_call(
        _body,
        grid=(NBLK + 1,),
        in_specs=[
            pl.BlockSpec((BLK, DIM), lambda i: (jnp.minimum(i, NBLK - 1), 0)),
        ],
        out_specs=pl.BlockSpec((BANK, DIM), lambda i: (0, 0)),
        out_shape=jax.ShapeDtypeStruct((BANK, DIM), jnp.float32),
        scratch_shapes=[
            pltpu.VMEM((BANK, DIM), jnp.float32),
            pltpu.VMEM((BANK, DIM), jnp.bfloat16),
            pltpu.VMEM((BANK, DIM), jnp.float32),
            pltpu.VMEM((1, BANK), jnp.float32),
            pltpu.VMEM((BLK, DIM), jnp.bfloat16),
            pltpu.VMEM((BLK, BANK), jnp.bfloat16),
        ],
    )(emb)


# speedup vs baseline: 1.0154x; 1.0154x over previous
import jax
import jax.numpy as jnp
from jax.experimental import pallas as pl
from jax.experimental.pallas import tpu as pltpu

BANK = 512
DIM = 768
EPSV = 1e-6
MOM = 0.9
BLK = 4096
NBLK = 16384 // BLK


def _norm_rows(x):
    n = jnp.sqrt(jnp.sum(x * x, axis=1, keepdims=True))
    return x / jnp.maximum(n, EPSV)


def _body(emb_ref, out_ref, pn_ref, pnb_ref, sums_ref, counts_ref,
          ebp_ref, ohwp_ref):
    i = pl.program_id(0)

    @pl.when(i == 0)
    def _init():
        pn = _norm_rows(_norm_rows(emb_ref[:BANK]))
        pn_ref[...] = pn
        pnb_ref[...] = pn.astype(jnp.bfloat16)
        sums_ref[...] = jnp.zeros_like(sums_ref)
        counts_ref[...] = jnp.zeros_like(counts_ref)
        ebp_ref[...] = jnp.zeros_like(ebp_ref)
        ohwp_ref[...] = jnp.zeros_like(ohwp_ref)

    sums_ref[...] += jax.lax.dot_general(
        ohwp_ref[...], ebp_ref[...], (((0,), (0,)), ((), ())),
        preferred_element_type=jnp.float32,
    )

    e = emb_ref[...]
    eb = e.astype(jnp.bfloat16)
    n2 = jax.lax.dot_general(
        eb * eb, jnp.ones((DIM, 1), jnp.bfloat16), (((1,), (0,)), ((), ())),
        preferred_element_type=jnp.float32,
    )
    w = jax.lax.rsqrt(jnp.maximum(n2, EPSV * EPSV))
    s = jax.lax.dot_general(
        eb, pnb_ref[...], (((1,), (1,)), ((), ())),
        preferred_element_type=jnp.float32,
    )
    m = jnp.max(s, axis=1, keepdims=True)
    first = jnp.where(i == 0, BANK, 0)
    live = jax.lax.broadcasted_iota(jnp.int32, (BLK, 1), 0) >= first
    m = jnp.where(live, m, jnp.inf)
    hit = s >= m
    onehotw = jnp.where(hit, w, 0.0).astype(jnp.bfloat16)
    ebp_ref[...] = eb
    ohwp_ref[...] = onehotw
    cnt = jnp.sum(hit.astype(jnp.float32), axis=0, keepdims=True)
    counts_ref[...] += jnp.where(i < NBLK, cnt, 0.0)

    @pl.when(i == NBLK)
    def _fin():
        ident = (jax.lax.broadcasted_iota(jnp.int32, (BANK, BANK), 0)
                 == jax.lax.broadcasted_iota(jnp.int32, (BANK, BANK), 1)
                 ).astype(jnp.float32)
        counts = jax.lax.dot_general(
            ident, counts_ref[...], (((1,), (1,)), ((), ())),
            preferred_element_type=jnp.float32,
        )
        means = sums_ref[...] / jnp.maximum(counts, 1.0)
        pn = pn_ref[...]
        upd = MOM * pn + (1.0 - MOM) * means
        updn = _norm_rows(upd)
        out_ref[...] = jnp.where(counts > 0.0, updn, pn)


def kernel(embeddings):
    emb = embeddings.astype(jnp.float32)
    return pl.pallas_call(
        _body,
        grid=(NBLK + 1,),
        in_specs=[
            pl.BlockSpec((BLK, DIM), lambda i: (jnp.minimum(i, NBLK - 1), 0)),
        ],
        out_specs=pl.BlockSpec((BANK, DIM), lambda i: (0, 0)),
        out_shape=jax.ShapeDtypeStruct((BANK, DIM), jnp.float32),
        scratch_shapes=[
            pltpu.VMEM((BANK, DIM), jnp.float32),
            pltpu.VMEM((BANK, DIM), jnp.bfloat16),
            pltpu.VMEM((BANK, DIM), jnp.float32),
            pltpu.VMEM((1, BANK), jnp.float32),
            pltpu.VMEM((BLK, DIM), jnp.bfloat16),
            pltpu.VMEM((BLK, BANK), jnp.bfloat16),
        ],
    )(emb)


# speedup vs baseline: 1.1357x; 1.1184x over previous
import jax
import jax.numpy as jnp
from jax.experimental import pallas as pl
from jax.experimental.pallas import tpu as pltpu

BANK = 512
DIM = 768
EPSV = 1e-6
MOM = 0.9
BLK = 2048
NBLK = 16384 // BLK


def _norm_rows(x):
    n = jnp.sqrt(jnp.sum(x * x, axis=1, keepdims=True))
    return x / jnp.maximum(n, EPSV)


def _body(emb_ref, out_ref, pn_ref, pnb_ref, sums_ref, counts_ref,
          ebp_ref, ohwp_ref):
    i = pl.program_id(0)

    @pl.when(i == 0)
    def _init():
        pn = _norm_rows(_norm_rows(emb_ref[:BANK]))
        pn_ref[...] = pn
        pnb_ref[...] = pn.astype(jnp.bfloat16)
        sums_ref[...] = jnp.zeros_like(sums_ref)
        counts_ref[...] = jnp.zeros_like(counts_ref)
        ebp_ref[...] = jnp.zeros_like(ebp_ref)
        ohwp_ref[...] = jnp.zeros_like(ohwp_ref)

    sums_ref[...] += jax.lax.dot_general(
        ohwp_ref[...], ebp_ref[...], (((0,), (0,)), ((), ())),
        preferred_element_type=jnp.float32,
    )

    e = emb_ref[...]
    eb = e.astype(jnp.bfloat16)
    n2 = jax.lax.dot_general(
        eb * eb, jnp.ones((DIM, 1), jnp.bfloat16), (((1,), (0,)), ((), ())),
        preferred_element_type=jnp.float32,
    )
    w = jax.lax.rsqrt(jnp.maximum(n2, EPSV * EPSV))
    s = jax.lax.dot_general(
        eb, pnb_ref[...], (((1,), (1,)), ((), ())),
        preferred_element_type=jnp.float32,
    )
    m = jnp.max(s, axis=1, keepdims=True)
    first = jnp.where(i == 0, BANK, 0)
    live = jax.lax.broadcasted_iota(jnp.int32, (BLK, 1), 0) >= first
    m = jnp.where(live, m, jnp.inf)
    hit = s >= m
    onehotw = jnp.where(hit, w, 0.0).astype(jnp.bfloat16)
    ebp_ref[...] = eb
    ohwp_ref[...] = onehotw
    cnt = jnp.sum(hit.astype(jnp.float32), axis=0, keepdims=True)
    counts_ref[...] += jnp.where(i < NBLK, cnt, 0.0)

    @pl.when(i == NBLK)
    def _fin():
        ident = (jax.lax.broadcasted_iota(jnp.int32, (BANK, BANK), 0)
                 == jax.lax.broadcasted_iota(jnp.int32, (BANK, BANK), 1)
                 ).astype(jnp.float32)
        counts = jax.lax.dot_general(
            ident, counts_ref[...], (((1,), (1,)), ((), ())),
            preferred_element_type=jnp.float32,
        )
        means = sums_ref[...] / jnp.maximum(counts, 1.0)
        pn = pn_ref[...]
        upd = MOM * pn + (1.0 - MOM) * means
        updn = _norm_rows(upd)
        out_ref[...] = jnp.where(counts > 0.0, updn, pn)


def kernel(embeddings):
    emb = embeddings.astype(jnp.float32)
    return pl.pallas_call(
        _body,
        grid=(NBLK + 1,),
        in_specs=[
            pl.BlockSpec((BLK, DIM), lambda i: (jnp.minimum(i, NBLK - 1), 0)),
        ],
        out_specs=pl.BlockSpec((BANK, DIM), lambda i: (0, 0)),
        out_shape=jax.ShapeDtypeStruct((BANK, DIM), jnp.float32),
        scratch_shapes=[
            pltpu.VMEM((BANK, DIM), jnp.float32),
            pltpu.VMEM((BANK, DIM), jnp.bfloat16),
            pltpu.VMEM((BANK, DIM), jnp.float32),
            pltpu.VMEM((1, BANK), jnp.float32),
            pltpu.VMEM((BLK, DIM), jnp.bfloat16),
            pltpu.VMEM((BLK, BANK), jnp.bfloat16),
        ],
    )(emb)


# speedup vs baseline: 1.1725x; 1.0324x over previous
import jax
import jax.numpy as jnp
from jax.experimental import pallas as pl
from jax.experimental.pallas import tpu as pltpu

BANK = 512
DIM = 768
EPSV = 1e-6
MOM = 0.9
BLK = 2048
NBLK = 16384 // BLK


def _norm_rows(x):
    n = jnp.sqrt(jnp.sum(x * x, axis=1, keepdims=True))
    return x / jnp.maximum(n, EPSV)


def _body(emb_ref, out_ref, pn_ref, pnb_ref, sums_ref, counts_ref,
          ebp_ref, ohwp_ref):
    i = pl.program_id(0)

    @pl.when(i == 0)
    def _init():
        pn = _norm_rows(_norm_rows(emb_ref[:BANK]))
        pn_ref[...] = pn
        pnb_ref[...] = pn.astype(jnp.bfloat16)
        sums_ref[...] = jnp.zeros_like(sums_ref)
        counts_ref[...] = jnp.zeros_like(counts_ref)
        ebp_ref[...] = jnp.zeros_like(ebp_ref)
        ohwp_ref[...] = jnp.zeros_like(ohwp_ref)

    sums_ref[...] += jax.lax.dot_general(
        ohwp_ref[...], ebp_ref[...], (((0,), (0,)), ((), ())),
        preferred_element_type=jnp.float32,
    )

    @pl.when(i < NBLK)
    def _compute():
        e = emb_ref[...]
        eb = e.astype(jnp.bfloat16)
        n2 = jax.lax.dot_general(
            eb * eb, jnp.ones((DIM, 1), jnp.bfloat16),
            (((1,), (0,)), ((), ())),
            preferred_element_type=jnp.float32,
        )
        w = jax.lax.rsqrt(jnp.maximum(n2, EPSV * EPSV))
        s = jax.lax.dot_general(
            eb, pnb_ref[...], (((1,), (1,)), ((), ())),
            preferred_element_type=jnp.float32,
        )
        m = jnp.max(s, axis=1, keepdims=True)
        first = jnp.where(i == 0, BANK, 0)
        live = jax.lax.broadcasted_iota(jnp.int32, (BLK, 1), 0) >= first
        m = jnp.where(live, m, jnp.inf)
        hit = s >= m
        onehotw = jnp.where(hit, w, 0.0).astype(jnp.bfloat16)
        ebp_ref[...] = eb
        ohwp_ref[...] = onehotw
        counts_ref[...] += jnp.sum(hit.astype(jnp.float32), axis=0,
                                   keepdims=True)

    @pl.when(i == NBLK)
    def _fin():
        ident = (jax.lax.broadcasted_iota(jnp.int32, (BANK, BANK), 0)
                 == jax.lax.broadcasted_iota(jnp.int32, (BANK, BANK), 1)
                 ).astype(jnp.float32)
        counts = jax.lax.dot_general(
            ident, counts_ref[...], (((1,), (1,)), ((), ())),
            preferred_element_type=jnp.float32,
        )
        means = sums_ref[...] / jnp.maximum(counts, 1.0)
        pn = pn_ref[...]
        upd = MOM * pn + (1.0 - MOM) * means
        updn = _norm_rows(upd)
        out_ref[...] = jnp.where(counts > 0.0, updn, pn)


def kernel(embeddings):
    emb = embeddings.astype(jnp.float32)
    return pl.pallas_call(
        _body,
        grid=(NBLK + 1,),
        in_specs=[
            pl.BlockSpec((BLK, DIM), lambda i: (jnp.minimum(i, NBLK - 1), 0)),
        ],
        out_specs=pl.BlockSpec((BANK, DIM), lambda i: (0, 0)),
        out_shape=jax.ShapeDtypeStruct((BANK, DIM), jnp.float32),
        scratch_shapes=[
            pltpu.VMEM((BANK, DIM), jnp.float32),
            pltpu.VMEM((BANK, DIM), jnp.bfloat16),
            pltpu.VMEM((BANK, DIM), jnp.float32),
            pltpu.VMEM((1, BANK), jnp.float32),
            pltpu.VMEM((BLK, DIM), jnp.bfloat16),
            pltpu.VMEM((BLK, BANK), jnp.bfloat16),
        ],
    )(emb)


# speedup vs baseline: 1.2271x; 1.0465x over previous
import jax
import jax.numpy as jnp
from jax.experimental import pallas as pl
from jax.experimental.pallas import tpu as pltpu

BANK = 512
DIM = 768
EPSV = 1e-6
MOM = 0.9
BLK = 2048
NBLK = 16384 // BLK


def _norm_rows(x):
    n = jnp.sqrt(jnp.sum(x * x, axis=1, keepdims=True))
    return x / jnp.maximum(n, EPSV)


def _body(emb_ref, out_ref, pn_ref, pnb_ref, sums_ref, counts_ref,
          ebp_ref, ohwp_ref):
    i = pl.program_id(0)

    @pl.when(i == 0)
    def _init():
        pn = _norm_rows(_norm_rows(emb_ref[:BANK]))
        pn_ref[...] = pn
        pnb_ref[...] = pn.astype(jnp.bfloat16)
        sums_ref[...] = jnp.zeros_like(sums_ref)
        counts_ref[...] = jnp.zeros_like(counts_ref)

    @pl.when(i > 0)
    def _acc():
        sums_ref[...] += jax.lax.dot_general(
            ohwp_ref[...], ebp_ref[...], (((0,), (0,)), ((), ())),
            preferred_element_type=jnp.float32,
        )

    @pl.when(i < NBLK)
    def _compute():
        e = emb_ref[...]
        eb = e.astype(jnp.bfloat16)
        n2 = jax.lax.dot_general(
            eb * eb, jnp.ones((DIM, 1), jnp.bfloat16),
            (((1,), (0,)), ((), ())),
            preferred_element_type=jnp.float32,
        )
        w = jax.lax.rsqrt(jnp.maximum(n2, EPSV * EPSV))
        s = jax.lax.dot_general(
            eb, pnb_ref[...], (((1,), (1,)), ((), ())),
            preferred_element_type=jnp.float32,
        )
        m = jnp.max(s, axis=1, keepdims=True)
        first = jnp.where(i == 0, BANK, 0)
        live = jax.lax.broadcasted_iota(jnp.int32, (BLK, 1), 0) >= first
        m = jnp.where(live, m, jnp.inf)
        hit = s >= m
        onehotw = jnp.where(hit, w, 0.0).astype(jnp.bfloat16)
        ebp_ref[...] = eb
        ohwp_ref[...] = onehotw
        counts_ref[...] += jnp.sum(hit.astype(jnp.float32), axis=0,
                                   keepdims=True)

    @pl.when(i == NBLK)
    def _fin():
        ident = (jax.lax.broadcasted_iota(jnp.int32, (BANK, BANK), 0)
                 == jax.lax.broadcasted_iota(jnp.int32, (BANK, BANK), 1)
                 ).astype(jnp.float32)
        counts = jax.lax.dot_general(
            ident, counts_ref[...], (((1,), (1,)), ((), ())),
            preferred_element_type=jnp.float32,
        )
        means = sums_ref[...] / jnp.maximum(counts, 1.0)
        pn = pn_ref[...]
        upd = MOM * pn + (1.0 - MOM) * means
        updn = _norm_rows(upd)
        out_ref[...] = jnp.where(counts > 0.0, updn, pn)


def kernel(embeddings):
    emb = embeddings.astype(jnp.float32)
    return pl.pallas_call(
        _body,
        grid=(NBLK + 1,),
        in_specs=[
            pl.BlockSpec((BLK, DIM), lambda i: (jnp.minimum(i, NBLK - 1), 0)),
        ],
        out_specs=pl.BlockSpec((BANK, DIM), lambda i: (0, 0)),
        out_shape=jax.ShapeDtypeStruct((BANK, DIM), jnp.float32),
        scratch_shapes=[
            pltpu.VMEM((BANK, DIM), jnp.float32),
            pltpu.VMEM((BANK, DIM), jnp.bfloat16),
            pltpu.VMEM((BANK, DIM), jnp.float32),
            pltpu.VMEM((1, BANK), jnp.float32),
            pltpu.VMEM((BLK, DIM), jnp.bfloat16),
            pltpu.VMEM((BLK, BANK), jnp.bfloat16),
        ],
    )(emb)
